# baseline (device time: 1276996 ns/iter reference)
import numpy as np
import jax
import jax.numpy as jnp
from jax import lax
from jax.experimental import pallas as pl
from jax.experimental.pallas import tpu as pltpu

N_DEV = 32
DH = 64


def kernel(x, Wq, Wk, Wv, Wo):
    B, Sq, D = x.shape
    C = Wq.shape[1]
    Hl = C // DH
    R = B * Sq

    x2 = x.reshape(R, D)
    w_all = jnp.concatenate([Wq, Wk, Wv, Wo.T], axis=0)

    inv = 1.0 / (10000.0 ** (np.arange(0, DH, 2) / DH))
    pos = np.arange(Sq)[:, None] * inv[None, :]
    cos = np.repeat(np.cos(pos), 2, axis=-1).astype(np.float32)
    sin = np.repeat(np.sin(pos), 2, axis=-1).astype(np.float32)
    cosT = jnp.asarray(np.tile(cos, (B, Hl)))
    sinT = jnp.asarray(np.tile(sin, (B, Hl)))
    P = np.zeros((C, C), np.float32)
    idx = np.arange(0, C, 2)
    P[idx + 1, idx] = -1.0
    P[idx, idx + 1] = 1.0
    P = jnp.asarray(P)

    def body(x_ref, w_ref, cos_ref, sin_ref, p_ref, out_ref,
             comm, ctx_ref, send_sems, recv_sems, credit_sem):
        my = lax.axis_index("i")
        left = lax.rem(my - 1 + N_DEV, N_DEV)
        right = lax.rem(my + 1, N_DEV)

        barrier = pltpu.get_barrier_semaphore()
        for nbr in (left, right):
            pl.semaphore_signal(barrier, inc=1, device_id=(nbr,),
                                device_id_type=pl.DeviceIdType.MESH)
        pl.semaphore_wait(barrier, 2)

        comm[0] = w_ref[...]
        out_ref[...] = jnp.zeros_like(out_ref)

        xloc = x_ref[...]
        cosV = cos_ref[...]
        sinV = sin_ref[...]
        pV = p_ref[...]

        def hop(h, carry):
            cur = lax.rem(h, 2)
            nxt = 1 - cur
            rdma = pltpu.make_async_remote_copy(
                src_ref=comm.at[cur],
                dst_ref=comm.at[nxt],
                send_sem=send_sems.at[cur],
                recv_sem=recv_sems.at[nxt],
                device_id=(right,),
                device_id_type=pl.DeviceIdType.MESH,
            )

            @pl.when(h < N_DEV - 1)
            def _():
                @pl.when(h >= 1)
                def _():
                    pl.semaphore_wait(credit_sem, 1)
                rdma.start()

            W = comm[cur]
            q = jnp.dot(xloc, W[0:D], preferred_element_type=jnp.float32)
            k = jnp.dot(xloc, W[D:2 * D], preferred_element_type=jnp.float32)
            v = jnp.dot(xloc, W[2 * D:3 * D], preferred_element_type=jnp.float32)
            wo_t = W[3 * D:4 * D]
            q = q * cosV + jnp.dot(q, pV, preferred_element_type=jnp.float32) * sinV
            k = k * cosV + jnp.dot(k, pV, preferred_element_type=jnp.float32) * sinV
            for b in range(B):
                for t in range(Hl):
                    rs = slice(b * Sq, (b + 1) * Sq)
                    cs = slice(t * DH, (t + 1) * DH)
                    qb, kb, vb = q[rs, cs], k[rs, cs], v[rs, cs]
                    s = lax.dot_general(qb, kb, (((1,), (1,)), ((), ())),
                                        preferred_element_type=jnp.float32)
                    s = s * 0.125
                    m = jnp.max(s, axis=-1, keepdims=True)
                    e = jnp.exp(s - m)
                    wgt = e / jnp.sum(e, axis=-1, keepdims=True)
                    ctx_ref[rs, cs] = jnp.dot(wgt, vb,
                                              preferred_element_type=jnp.float32)
            out_ref[...] += lax.dot_general(
                ctx_ref[...], wo_t, (((1,), (1,)), ((), ())),
                preferred_element_type=jnp.float32)

            @pl.when(h < N_DEV - 1)
            def _():
                rdma.wait()
                pl.semaphore_signal(credit_sem, inc=1, device_id=(left,),
                                    device_id_type=pl.DeviceIdType.MESH)
            return carry

        lax.fori_loop(0, N_DEV, hop, 0)
        pl.semaphore_wait(credit_sem, 1)

    out = pl.pallas_call(
        body,
        out_shape=jax.ShapeDtypeStruct((R, D), jnp.float32),
        in_specs=[pl.BlockSpec(memory_space=pltpu.VMEM)] * 5,
        out_specs=pl.BlockSpec(memory_space=pltpu.VMEM),
        scratch_shapes=[
            pltpu.VMEM((2, 4 * D, C), jnp.float32),
            pltpu.VMEM((R, C), jnp.float32),
            pltpu.SemaphoreType.DMA((2,)),
            pltpu.SemaphoreType.DMA((2,)),
            pltpu.SemaphoreType.REGULAR,
        ],
        compiler_params=pltpu.CompilerParams(collective_id=0),
    )(x2, w_all, cosT, sinT, P)
    return out.reshape(B, Sq, D)


# device time: 1155132 ns/iter; 1.1055x vs baseline; 1.1055x over previous
import numpy as np
import jax
import jax.numpy as jnp
from jax import lax
from jax.experimental import pallas as pl
from jax.experimental.pallas import tpu as pltpu

N_DEV = 32
DH = 64
N_R = N_DEV // 2
N_L = N_DEV - 1 - N_R


def kernel(x, Wq, Wk, Wv, Wo):
    B, Sq, D = x.shape
    C = Wq.shape[1]
    Hl = C // DH
    R = B * Sq

    x2 = x.reshape(R, D)
    w_all = jnp.concatenate([Wq, Wk, Wv, Wo.T], axis=0)

    inv = 1.0 / (10000.0 ** (np.arange(0, DH, 2) / DH))
    pos = np.arange(Sq)[:, None] * inv[None, :]
    cos = np.repeat(np.cos(pos), 2, axis=-1).astype(np.float32)
    sin = np.repeat(np.sin(pos), 2, axis=-1).astype(np.float32)
    cosT = jnp.asarray(np.tile(cos, (B, Hl)))
    sinT = jnp.asarray(np.tile(sin, (B, Hl)))
    P = np.zeros((C, C), np.float32)
    idx = np.arange(0, C, 2)
    P[idx + 1, idx] = -1.0
    P[idx, idx + 1] = 1.0
    P = jnp.asarray(P)

    def body(x_ref, w_ref, cos_ref, sin_ref, p_ref, out_ref,
             comm_r, comm_l, ctx_ref,
             ss_r, rs_r, ss_l, rs_l, credit_r, credit_l):
        my = lax.axis_index("i")
        left = lax.rem(my - 1 + N_DEV, N_DEV)
        right = lax.rem(my + 1, N_DEV)

        barrier = pltpu.get_barrier_semaphore()
        for nbr in (left, right):
            pl.semaphore_signal(barrier, inc=1, device_id=(nbr,),
                                device_id_type=pl.DeviceIdType.MESH)
        pl.semaphore_wait(barrier, 2)

        comm_r[0] = w_ref[...]
        comm_l[0] = w_ref[...]
        out_ref[...] = jnp.zeros_like(out_ref)

        xloc = x_ref[...]
        cosV = cos_ref[...]
        sinV = sin_ref[...]
        pV = p_ref[...]

        def compute_chunk(W):
            q = jnp.dot(xloc, W[0:D], preferred_element_type=jnp.float32)
            k = jnp.dot(xloc, W[D:2 * D], preferred_element_type=jnp.float32)
            v = jnp.dot(xloc, W[2 * D:3 * D], preferred_element_type=jnp.float32)
            wo_t = W[3 * D:4 * D]
            q = q * cosV + jnp.dot(q, pV, preferred_element_type=jnp.float32) * sinV
            k = k * cosV + jnp.dot(k, pV, preferred_element_type=jnp.float32) * sinV
            for b in range(B):
                for t in range(Hl):
                    rs = slice(b * Sq, (b + 1) * Sq)
                    cs = slice(t * DH, (t + 1) * DH)
                    qb, kb, vb = q[rs, cs], k[rs, cs], v[rs, cs]
                    s = lax.dot_general(qb, kb, (((1,), (1,)), ((), ())),
                                        preferred_element_type=jnp.float32)
                    s = s * 0.125
                    m = jnp.max(s, axis=-1, keepdims=True)
                    e = jnp.exp(s - m)
                    wgt = e / jnp.sum(e, axis=-1, keepdims=True)
                    ctx_ref[rs, cs] = jnp.dot(wgt, vb,
                                              preferred_element_type=jnp.float32)
            out_ref[...] += lax.dot_general(
                ctx_ref[...], wo_t, (((1,), (1,)), ((), ())),
                preferred_element_type=jnp.float32)

        def step(s, carry):
            cur = lax.rem(s, 2)
            nxt = 1 - cur
            rd_r = pltpu.make_async_remote_copy(
                src_ref=comm_r.at[cur], dst_ref=comm_r.at[nxt],
                send_sem=ss_r.at[cur], recv_sem=rs_r.at[nxt],
                device_id=(right,), device_id_type=pl.DeviceIdType.MESH)
            rd_l = pltpu.make_async_remote_copy(
                src_ref=comm_l.at[cur], dst_ref=comm_l.at[nxt],
                send_sem=ss_l.at[cur], recv_sem=rs_l.at[nxt],
                device_id=(left,), device_id_type=pl.DeviceIdType.MESH)

            @pl.when(s < N_R)
            def _():
                @pl.when(s >= 1)
                def _():
                    pl.semaphore_wait(credit_r, 1)
                rd_r.start()

            @pl.when(s < N_L)
            def _():
                @pl.when(s >= 1)
                def _():
                    pl.semaphore_wait(credit_l, 1)
                rd_l.start()

            compute_chunk(comm_r[cur])
            @pl.when((s >= 1) & (s <= N_L))
            def _():
                compute_chunk(comm_l[cur])

            @pl.when(s < N_R)
            def _():
                rd_r.wait()
                pl.semaphore_signal(credit_r, inc=1, device_id=(left,),
                                    device_id_type=pl.DeviceIdType.MESH)

            @pl.when(s < N_L)
            def _():
                rd_l.wait()
                pl.semaphore_signal(credit_l, inc=1, device_id=(right,),
                                    device_id_type=pl.DeviceIdType.MESH)
            return carry

        lax.fori_loop(0, N_R + 1, step, 0)
        pl.semaphore_wait(credit_r, 1)
        pl.semaphore_wait(credit_l, 1)

    out = pl.pallas_call(
        body,
        out_shape=jax.ShapeDtypeStruct((R, D), jnp.float32),
        in_specs=[pl.BlockSpec(memory_space=pltpu.VMEM)] * 5,
        out_specs=pl.BlockSpec(memory_space=pltpu.VMEM),
        scratch_shapes=[
            pltpu.VMEM((2, 4 * D, C), jnp.float32),
            pltpu.VMEM((2, 4 * D, C), jnp.float32),
            pltpu.VMEM((R, C), jnp.float32),
            pltpu.SemaphoreType.DMA((2,)),
            pltpu.SemaphoreType.DMA((2,)),
            pltpu.SemaphoreType.DMA((2,)),
            pltpu.SemaphoreType.DMA((2,)),
            pltpu.SemaphoreType.REGULAR,
            pltpu.SemaphoreType.REGULAR,
        ],
        compiler_params=pltpu.CompilerParams(collective_id=0),
    )(x2, w_all, cosT, sinT, P)
    return out.reshape(B, Sq, D)


# device time: 634316 ns/iter; 2.0132x vs baseline; 1.8211x over previous
import numpy as np
import jax
import jax.numpy as jnp
from jax import lax
from jax.experimental import pallas as pl
from jax.experimental.pallas import tpu as pltpu

N_DEV = 32
DH = 64
N_R = N_DEV // 2
N_L = N_DEV - 1 - N_R


def kernel(x, Wq, Wk, Wv, Wo):
    B, Sq, D = x.shape
    C = Wq.shape[1]
    Hl = C // DH
    R = B * Sq

    x2 = x.reshape(R, D).astype(jnp.bfloat16)
    w_all = jnp.concatenate([Wq, Wk, Wv, Wo.T], axis=1).astype(jnp.bfloat16)

    inv = 1.0 / (10000.0 ** (np.arange(0, DH, 2) / DH))
    pos = np.arange(Sq)[:, None] * inv[None, :]
    cos = np.repeat(np.cos(pos), 2, axis=-1).astype(np.float32)
    sin = np.repeat(np.sin(pos), 2, axis=-1).astype(np.float32)
    cosT = jnp.asarray(np.tile(cos, (B, Hl)))
    sinT = jnp.asarray(np.tile(sin, (B, Hl)))
    P = np.zeros((C, C), np.float32)
    idx = np.arange(0, C, 2)
    P[idx + 1, idx] = -1.0
    P[idx, idx + 1] = 1.0
    P = jnp.asarray(P.astype(np.float32)).astype(jnp.bfloat16)

    def body(x_ref, w_ref, cos_ref, sin_ref, p_ref, out_ref,
             comm_r, comm_l, ctx_ref,
             ss_r, rs_r, ss_l, rs_l, credit_r, credit_l):
        my = lax.axis_index("i")
        left = lax.rem(my - 1 + N_DEV, N_DEV)
        right = lax.rem(my + 1, N_DEV)

        barrier = pltpu.get_barrier_semaphore()
        for nbr in (left, right):
            pl.semaphore_signal(barrier, inc=1, device_id=(nbr,),
                                device_id_type=pl.DeviceIdType.MESH)
        pl.semaphore_wait(barrier, 2)

        comm_r[0] = w_ref[...]
        comm_l[0] = w_ref[...]
        out_ref[...] = jnp.zeros_like(out_ref)

        xloc = x_ref[...]
        cosV = cos_ref[...]
        sinV = sin_ref[...]
        pV = p_ref[...]

        def compute_chunk(Wc):
            qkv = jnp.dot(xloc, Wc[:, 0:3 * C],
                          preferred_element_type=jnp.float32)
            wo_t = Wc[:, 3 * C:4 * C]
            q = qkv[:, 0:C]
            k = qkv[:, C:2 * C]
            v16 = qkv[:, 2 * C:3 * C].astype(jnp.bfloat16)
            qr = jnp.dot(q.astype(jnp.bfloat16), pV,
                         preferred_element_type=jnp.float32)
            kr = jnp.dot(k.astype(jnp.bfloat16), pV,
                         preferred_element_type=jnp.float32)
            q16 = (q * cosV + qr * sinV).astype(jnp.bfloat16)
            k16 = (k * cosV + kr * sinV).astype(jnp.bfloat16)
            for b in range(B):
                for t in range(Hl):
                    rs = slice(b * Sq, (b + 1) * Sq)
                    cs = slice(t * DH, (t + 1) * DH)
                    s = lax.dot_general(q16[rs, cs], k16[rs, cs],
                                        (((1,), (1,)), ((), ())),
                                        preferred_element_type=jnp.float32)
                    s = s * 0.125
                    m = jnp.max(s, axis=-1, keepdims=True)
                    e = jnp.exp(s - m)
                    wgt = (e / jnp.sum(e, axis=-1, keepdims=True)
                           ).astype(jnp.bfloat16)
                    ctx_ref[rs, cs] = jnp.dot(
                        wgt, v16[rs, cs],
                        preferred_element_type=jnp.float32).astype(jnp.bfloat16)
            out_ref[...] += lax.dot_general(
                ctx_ref[...], wo_t, (((1,), (1,)), ((), ())),
                preferred_element_type=jnp.float32)

        def step(s, carry):
            cur = lax.rem(s, 2)
            nxt = 1 - cur
            rd_r = pltpu.make_async_remote_copy(
                src_ref=comm_r.at[cur], dst_ref=comm_r.at[nxt],
                send_sem=ss_r.at[cur], recv_sem=rs_r.at[nxt],
                device_id=(right,), device_id_type=pl.DeviceIdType.MESH)
            rd_l = pltpu.make_async_remote_copy(
                src_ref=comm_l.at[cur], dst_ref=comm_l.at[nxt],
                send_sem=ss_l.at[cur], recv_sem=rs_l.at[nxt],
                device_id=(left,), device_id_type=pl.DeviceIdType.MESH)

            @pl.when(s < N_R)
            def _():
                @pl.when(s >= 1)
                def _():
                    pl.semaphore_wait(credit_r, 1)
                rd_r.start()

            @pl.when(s < N_L)
            def _():
                @pl.when(s >= 1)
                def _():
                    pl.semaphore_wait(credit_l, 1)
                rd_l.start()

            compute_chunk(comm_r[cur])
            @pl.when((s >= 1) & (s <= N_L))
            def _():
                compute_chunk(comm_l[cur])

            @pl.when(s < N_R)
            def _():
                rd_r.wait()
                pl.semaphore_signal(credit_r, inc=1, device_id=(left,),
                                    device_id_type=pl.DeviceIdType.MESH)

            @pl.when(s < N_L)
            def _():
                rd_l.wait()
                pl.semaphore_signal(credit_l, inc=1, device_id=(right,),
                                    device_id_type=pl.DeviceIdType.MESH)
            return carry

        lax.fori_loop(0, N_R + 1, step, 0)
        pl.semaphore_wait(credit_r, 1)
        pl.semaphore_wait(credit_l, 1)

    out = pl.pallas_call(
        body,
        out_shape=jax.ShapeDtypeStruct((R, D), jnp.float32),
        in_specs=[pl.BlockSpec(memory_space=pltpu.VMEM)] * 5,
        out_specs=pl.BlockSpec(memory_space=pltpu.VMEM),
        scratch_shapes=[
            pltpu.VMEM((2, D, 4 * C), jnp.bfloat16),
            pltpu.VMEM((2, D, 4 * C), jnp.bfloat16),
            pltpu.VMEM((R, C), jnp.bfloat16),
            pltpu.SemaphoreType.DMA((2,)),
            pltpu.SemaphoreType.DMA((2,)),
            pltpu.SemaphoreType.DMA((2,)),
            pltpu.SemaphoreType.DMA((2,)),
            pltpu.SemaphoreType.REGULAR,
            pltpu.SemaphoreType.REGULAR,
        ],
        compiler_params=pltpu.CompilerParams(collective_id=0),
    )(x2, w_all, cosT, sinT, P)
    return out.reshape(B, Sq, D)


# device time: 323383 ns/iter; 3.9489x vs baseline; 1.9615x over previous
import numpy as np
import jax
import jax.numpy as jnp
from jax import lax
from jax.experimental import pallas as pl
from jax.experimental.pallas import tpu as pltpu

N_DEV = 32
DH = 64
N_R = N_DEV // 2
N_L = N_DEV - 1 - N_R


def _ring_tables():
    logical = []
    for z in range(4):
        for yi in range(4):
            row = [(x, yi, z) for x in range(2)]
            if yi % 2:
                row.reverse()
            logical.extend(row)
    l_of = {c: i for i, c in enumerate(logical)}
    path_yz = []
    for z in range(4):
        ys = range(4) if z % 2 == 0 else range(3, -1, -1)
        path_yz.extend((y, z) for y in ys)
    seq = [(0, y, z) for (y, z) in path_yz]
    seq += [(1, y, z) for (y, z) in reversed(path_yz)]
    cyc = [l_of[c] for c in seq]
    succ = np.empty(N_DEV, np.int32)
    pred = np.empty(N_DEV, np.int32)
    for m in range(N_DEV):
        succ[cyc[m]] = cyc[(m + 1) % N_DEV]
        pred[cyc[(m + 1) % N_DEV]] = cyc[m]
    return succ, pred


_SUCC, _PRED = _ring_tables()


def kernel(x, Wq, Wk, Wv, Wo):
    B, Sq, D = x.shape
    C = Wq.shape[1]
    Hl = C // DH
    R = B * Sq

    x2 = x.reshape(R, D).astype(jnp.bfloat16)
    w_all = jnp.concatenate([Wq, Wk, Wv, Wo.T], axis=1).astype(jnp.bfloat16)

    inv = 1.0 / (10000.0 ** (np.arange(0, DH, 2) / DH))
    pos = np.arange(Sq)[:, None] * inv[None, :]
    cos = np.repeat(np.cos(pos), 2, axis=-1).astype(np.float32)
    sin = np.repeat(np.sin(pos), 2, axis=-1).astype(np.float32)
    cosT = jnp.asarray(np.tile(cos, (B, Hl)))
    sinT = jnp.asarray(np.tile(sin, (B, Hl)))
    P = np.zeros((C, C), np.float32)
    idx = np.arange(0, C, 2)
    P[idx + 1, idx] = -1.0
    P[idx, idx + 1] = 1.0
    P = jnp.asarray(P.astype(np.float32)).astype(jnp.bfloat16)

    def body(succ_ref, pred_ref, x_ref, w_ref, cos_ref, sin_ref, p_ref,
             out_ref, comm_r, comm_l, ctx_ref,
             ss_r, rs_r, ss_l, rs_l, credit_r, credit_l):
        right = succ_ref[0]
        left = pred_ref[0]

        barrier = pltpu.get_barrier_semaphore()
        for nbr in (left, right):
            pl.semaphore_signal(barrier, inc=1, device_id=(nbr,),
                                device_id_type=pl.DeviceIdType.MESH)
        pl.semaphore_wait(barrier, 2)

        comm_r[0] = w_ref[...]
        comm_l[0] = w_ref[...]
        out_ref[...] = jnp.zeros_like(out_ref)

        xloc = x_ref[...]
        cosV = cos_ref[...]
        sinV = sin_ref[...]
        pV = p_ref[...]

        def compute_chunk(Wc):
            qkv = jnp.dot(xloc, Wc[:, 0:3 * C],
                          preferred_element_type=jnp.float32)
            wo_t = Wc[:, 3 * C:4 * C]
            q = qkv[:, 0:C]
            k = qkv[:, C:2 * C]
            v16 = qkv[:, 2 * C:3 * C].astype(jnp.bfloat16)
            qr = jnp.dot(q.astype(jnp.bfloat16), pV,
                         preferred_element_type=jnp.float32)
            kr = jnp.dot(k.astype(jnp.bfloat16), pV,
                         preferred_element_type=jnp.float32)
            q16 = (q * cosV + qr * sinV).astype(jnp.bfloat16)
            k16 = (k * cosV + kr * sinV).astype(jnp.bfloat16)
            for b in range(B):
                for t in range(Hl):
                    rs = slice(b * Sq, (b + 1) * Sq)
                    cs = slice(t * DH, (t + 1) * DH)
                    s = lax.dot_general(q16[rs, cs], k16[rs, cs],
                                        (((1,), (1,)), ((), ())),
                                        preferred_element_type=jnp.float32)
                    s = s * 0.125
                    m = jnp.max(s, axis=-1, keepdims=True)
                    e = jnp.exp(s - m)
                    wgt = (e / jnp.sum(e, axis=-1, keepdims=True)
                           ).astype(jnp.bfloat16)
                    ctx_ref[rs, cs] = jnp.dot(
                        wgt, v16[rs, cs],
                        preferred_element_type=jnp.float32).astype(jnp.bfloat16)
            out_ref[...] += lax.dot_general(
                ctx_ref[...], wo_t, (((1,), (1,)), ((), ())),
                preferred_element_type=jnp.float32)

        def step(s, carry):
            cur = lax.rem(s, 2)
            nxt = 1 - cur
            rd_r = pltpu.make_async_remote_copy(
                src_ref=comm_r.at[cur], dst_ref=comm_r.at[nxt],
                send_sem=ss_r.at[cur], recv_sem=rs_r.at[nxt],
                device_id=(right,), device_id_type=pl.DeviceIdType.MESH)
            rd_l = pltpu.make_async_remote_copy(
                src_ref=comm_l.at[cur], dst_ref=comm_l.at[nxt],
                send_sem=ss_l.at[cur], recv_sem=rs_l.at[nxt],
                device_id=(left,), device_id_type=pl.DeviceIdType.MESH)

            @pl.when(s < N_R)
            def _():
                @pl.when(s >= 1)
                def _():
                    pl.semaphore_wait(credit_r, 1)
                rd_r.start()

            @pl.when(s < N_L)
            def _():
                @pl.when(s >= 1)
                def _():
                    pl.semaphore_wait(credit_l, 1)
                rd_l.start()

            compute_chunk(comm_r[cur])
            @pl.when((s >= 1) & (s <= N_L))
            def _():
                compute_chunk(comm_l[cur])

            @pl.when(s < N_R)
            def _():
                rd_r.wait()
                pl.semaphore_signal(credit_r, inc=1, device_id=(left,),
                                    device_id_type=pl.DeviceIdType.MESH)

            @pl.when(s < N_L)
            def _():
                rd_l.wait()
                pl.semaphore_signal(credit_l, inc=1, device_id=(right,),
                                    device_id_type=pl.DeviceIdType.MESH)
            return carry

        lax.fori_loop(0, N_R + 1, step, 0)
        pl.semaphore_wait(credit_r, 1)
        pl.semaphore_wait(credit_l, 1)

    my = lax.axis_index("i")
    succ_s = jnp.asarray(_SUCC)[my].reshape(1)
    pred_s = jnp.asarray(_PRED)[my].reshape(1)

    out = pl.pallas_call(
        body,
        out_shape=jax.ShapeDtypeStruct((R, D), jnp.float32),
        in_specs=[pl.BlockSpec(memory_space=pltpu.SMEM)] * 2
        + [pl.BlockSpec(memory_space=pltpu.VMEM)] * 5,
        out_specs=pl.BlockSpec(memory_space=pltpu.VMEM),
        scratch_shapes=[
            pltpu.VMEM((2, D, 4 * C), jnp.bfloat16),
            pltpu.VMEM((2, D, 4 * C), jnp.bfloat16),
            pltpu.VMEM((R, C), jnp.bfloat16),
            pltpu.SemaphoreType.DMA((2,)),
            pltpu.SemaphoreType.DMA((2,)),
            pltpu.SemaphoreType.DMA((2,)),
            pltpu.SemaphoreType.DMA((2,)),
            pltpu.SemaphoreType.REGULAR,
            pltpu.SemaphoreType.REGULAR,
        ],
        compiler_params=pltpu.CompilerParams(collective_id=0),
    )(succ_s, pred_s, x2, w_all, cosT, sinT, P)
    return out.reshape(B, Sq, D)


# device time: 315787 ns/iter; 4.0439x vs baseline; 1.0241x over previous
import numpy as np
import jax
import jax.numpy as jnp
from jax import lax
from jax.experimental import pallas as pl
from jax.experimental.pallas import tpu as pltpu

N_DEV = 32
DH = 64
N_R = N_DEV // 2
N_L = N_DEV - 1 - N_R
NSLOT = 4


def _ring_tables():
    logical = []
    for z in range(4):
        for yi in range(4):
            row = [(x, yi, z) for x in range(2)]
            if yi % 2:
                row.reverse()
            logical.extend(row)
    l_of = {c: i for i, c in enumerate(logical)}
    path_yz = []
    for z in range(4):
        ys = range(4) if z % 2 == 0 else range(3, -1, -1)
        path_yz.extend((y, z) for y in ys)
    seq = [(0, y, z) for (y, z) in path_yz]
    seq += [(1, y, z) for (y, z) in reversed(path_yz)]
    cyc = [l_of[c] for c in seq]
    succ = np.empty(N_DEV, np.int32)
    pred = np.empty(N_DEV, np.int32)
    for m in range(N_DEV):
        succ[cyc[m]] = cyc[(m + 1) % N_DEV]
        pred[cyc[(m + 1) % N_DEV]] = cyc[m]
    return succ, pred


_SUCC, _PRED = _ring_tables()


def kernel(x, Wq, Wk, Wv, Wo):
    B, Sq, D = x.shape
    C = Wq.shape[1]
    Hl = C // DH
    R = B * Sq

    x2 = x.reshape(R, D).astype(jnp.bfloat16)
    w_all = jnp.concatenate([Wq, Wk, Wv, Wo.T], axis=1).astype(jnp.bfloat16)

    inv = 1.0 / (10000.0 ** (np.arange(0, DH, 2) / DH))
    pos = np.arange(Sq)[:, None] * inv[None, :]
    cos = np.repeat(np.cos(pos), 2, axis=-1).astype(np.float32)
    sin = np.repeat(np.sin(pos), 2, axis=-1).astype(np.float32)
    cosT = jnp.asarray(np.tile(cos, (B, Hl)))
    sinT = jnp.asarray(np.tile(sin, (B, Hl)))
    P = np.zeros((C, C), np.float32)
    idx = np.arange(0, C, 2)
    P[idx + 1, idx] = -1.0
    P[idx, idx + 1] = 1.0
    P = jnp.asarray(P.astype(np.float32)).astype(jnp.bfloat16)

    def body(succ_ref, pred_ref, x_ref, w_ref, cos_ref, sin_ref, p_ref,
             out_ref, comm_r, comm_l, ctx_ref,
             ss_r, rs_r, ss_l, rs_l, credit_r, credit_l):
        right = succ_ref[0]
        left = pred_ref[0]

        barrier = pltpu.get_barrier_semaphore()
        for nbr in (left, right):
            pl.semaphore_signal(barrier, inc=1, device_id=(nbr,),
                                device_id_type=pl.DeviceIdType.MESH)
        pl.semaphore_wait(barrier, 2)

        comm_r[0] = w_ref[...]
        comm_l[0] = w_ref[...]
        out_ref[...] = jnp.zeros_like(out_ref)

        xloc = x_ref[...]
        cosV = cos_ref[...]
        sinV = sin_ref[...]
        pV = p_ref[...]

        def compute_chunk(Wc):
            qkv = jnp.dot(xloc, Wc[:, 0:3 * C],
                          preferred_element_type=jnp.float32)
            wo_t = Wc[:, 3 * C:4 * C]
            q = qkv[:, 0:C]
            k = qkv[:, C:2 * C]
            v16 = qkv[:, 2 * C:3 * C].astype(jnp.bfloat16)
            qr = jnp.dot(q.astype(jnp.bfloat16), pV,
                         preferred_element_type=jnp.float32)
            kr = jnp.dot(k.astype(jnp.bfloat16), pV,
                         preferred_element_type=jnp.float32)
            q16 = (q * cosV + qr * sinV).astype(jnp.bfloat16)
            k16 = (k * cosV + kr * sinV).astype(jnp.bfloat16)
            for b in range(B):
                for t in range(Hl):
                    rs = slice(b * Sq, (b + 1) * Sq)
                    cs = slice(t * DH, (t + 1) * DH)
                    s = lax.dot_general(q16[rs, cs], k16[rs, cs],
                                        (((1,), (1,)), ((), ())),
                                        preferred_element_type=jnp.float32)
                    s = s * 0.125
                    m = jnp.max(s, axis=-1, keepdims=True)
                    e = jnp.exp(s - m)
                    wgt = (e / jnp.sum(e, axis=-1, keepdims=True)
                           ).astype(jnp.bfloat16)
                    ctx_ref[rs, cs] = jnp.dot(
                        wgt, v16[rs, cs],
                        preferred_element_type=jnp.float32).astype(jnp.bfloat16)
            out_ref[...] += lax.dot_general(
                ctx_ref[...], wo_t, (((1,), (1,)), ((), ())),
                preferred_element_type=jnp.float32)

        def step(s, carry):
            cur = lax.rem(s, NSLOT)
            nxt = lax.rem(s + 1, NSLOT)
            rd_r = pltpu.make_async_remote_copy(
                src_ref=comm_r.at[cur], dst_ref=comm_r.at[nxt],
                send_sem=ss_r.at[cur], recv_sem=rs_r.at[nxt],
                device_id=(right,), device_id_type=pl.DeviceIdType.MESH)
            rd_l = pltpu.make_async_remote_copy(
                src_ref=comm_l.at[cur], dst_ref=comm_l.at[nxt],
                send_sem=ss_l.at[cur], recv_sem=rs_l.at[nxt],
                device_id=(left,), device_id_type=pl.DeviceIdType.MESH)

            @pl.when(s < N_R)
            def _():
                @pl.when(s >= NSLOT - 1)
                def _():
                    pl.semaphore_wait(credit_r, 1)
                rd_r.start()

            @pl.when(s < N_L)
            def _():
                @pl.when(s >= NSLOT - 1)
                def _():
                    pl.semaphore_wait(credit_l, 1)
                rd_l.start()

            compute_chunk(comm_r[cur])
            @pl.when((s >= 1) & (s <= N_L))
            def _():
                compute_chunk(comm_l[cur])

            @pl.when(s < N_R)
            def _():
                rd_r.wait()
                pl.semaphore_signal(credit_r, inc=1, device_id=(left,),
                                    device_id_type=pl.DeviceIdType.MESH)

            @pl.when(s < N_L)
            def _():
                rd_l.wait()
                pl.semaphore_signal(credit_l, inc=1, device_id=(right,),
                                    device_id_type=pl.DeviceIdType.MESH)
            return carry

        lax.fori_loop(0, N_R + 1, step, 0)
        pl.semaphore_wait(credit_r, NSLOT - 1)
        pl.semaphore_wait(credit_l, NSLOT - 1)

    my = lax.axis_index("i")
    succ_s = jnp.asarray(_SUCC)[my].reshape(1)
    pred_s = jnp.asarray(_PRED)[my].reshape(1)

    out = pl.pallas_call(
        body,
        out_shape=jax.ShapeDtypeStruct((R, D), jnp.float32),
        in_specs=[pl.BlockSpec(memory_space=pltpu.SMEM)] * 2
        + [pl.BlockSpec(memory_space=pltpu.VMEM)] * 5,
        out_specs=pl.BlockSpec(memory_space=pltpu.VMEM),
        scratch_shapes=[
            pltpu.VMEM((NSLOT, D, 4 * C), jnp.bfloat16),
            pltpu.VMEM((NSLOT, D, 4 * C), jnp.bfloat16),
            pltpu.VMEM((R, C), jnp.bfloat16),
            pltpu.SemaphoreType.DMA((NSLOT,)),
            pltpu.SemaphoreType.DMA((NSLOT,)),
            pltpu.SemaphoreType.DMA((NSLOT,)),
            pltpu.SemaphoreType.DMA((NSLOT,)),
            pltpu.SemaphoreType.REGULAR,
            pltpu.SemaphoreType.REGULAR,
        ],
        compiler_params=pltpu.CompilerParams(collective_id=0),
    )(succ_s, pred_s, x2, w_all, cosT, sinT, P)
    return out.reshape(B, Sq, D)


# device time: 312969 ns/iter; 4.0803x vs baseline; 1.0090x over previous
import numpy as np
import jax
import jax.numpy as jnp
from jax import lax
from jax.experimental import pallas as pl
from jax.experimental.pallas import tpu as pltpu

N_DEV = 32
DH = 64
N_R = N_DEV // 2
N_L = N_DEV - 1 - N_R
NSLOT = 4


def _ring_tables():
    logical = []
    for z in range(4):
        for yi in range(4):
            row = [(x, yi, z) for x in range(2)]
            if yi % 2:
                row.reverse()
            logical.extend(row)
    l_of = {c: i for i, c in enumerate(logical)}
    path_yz = []
    for z in range(4):
        ys = range(4) if z % 2 == 0 else range(3, -1, -1)
        path_yz.extend((y, z) for y in ys)
    seq = [(0, y, z) for (y, z) in path_yz]
    seq += [(1, y, z) for (y, z) in reversed(path_yz)]
    cyc = [l_of[c] for c in seq]
    succ = np.empty(N_DEV, np.int32)
    pred = np.empty(N_DEV, np.int32)
    for m in range(N_DEV):
        succ[cyc[m]] = cyc[(m + 1) % N_DEV]
        pred[cyc[(m + 1) % N_DEV]] = cyc[m]
    return succ, pred


_SUCC, _PRED = _ring_tables()


def kernel(x, Wq, Wk, Wv, Wo):
    B, Sq, D = x.shape
    C = Wq.shape[1]
    Hl = C // DH
    R = B * Sq

    x2 = x.reshape(R, D).astype(jnp.bfloat16)
    w_all = jnp.concatenate([Wq, Wk, Wv, Wo.T], axis=1).astype(jnp.bfloat16)

    inv = 1.0 / (10000.0 ** (np.arange(0, DH, 2) / DH))
    pos = np.arange(Sq)[:, None] * inv[None, :]
    cos = np.repeat(np.cos(pos), 2, axis=-1).astype(np.float32)
    sin = np.repeat(np.sin(pos), 2, axis=-1).astype(np.float32)
    cosT = jnp.asarray(np.tile(cos, (B, Hl)))
    sinT = jnp.asarray(np.tile(sin, (B, Hl)))
    P = np.zeros((C, C), np.float32)
    idx = np.arange(0, C, 2)
    P[idx + 1, idx] = -1.0
    P[idx, idx + 1] = 1.0
    P = jnp.asarray(P.astype(np.float32)).astype(jnp.bfloat16)

    def body(succ_ref, pred_ref, x_ref, w_ref, cos_ref, sin_ref, p_ref,
             out_ref, comm_r, comm_l, ctx_ref,
             ss_r, rs_r, ss_l, rs_l, credit_r, credit_l):
        my = lax.axis_index("i")
        right = succ_ref[my]
        left = pred_ref[my]

        barrier = pltpu.get_barrier_semaphore()
        for nbr in (left, right):
            pl.semaphore_signal(barrier, inc=1, device_id=(nbr,),
                                device_id_type=pl.DeviceIdType.MESH)
        pl.semaphore_wait(barrier, 2)

        comm_r[0] = w_ref[...]
        comm_l[0] = w_ref[...]
        out_ref[...] = jnp.zeros_like(out_ref)

        xloc = x_ref[...]
        cosV = cos_ref[...]
        sinV = sin_ref[...]
        pV = p_ref[...]

        def compute_chunk(Wc):
            qkv = jnp.dot(xloc, Wc[:, 0:3 * C],
                          preferred_element_type=jnp.float32)
            wo_t = Wc[:, 3 * C:4 * C]
            q = qkv[:, 0:C]
            k = qkv[:, C:2 * C]
            v16 = qkv[:, 2 * C:3 * C].astype(jnp.bfloat16)
            qr = jnp.dot(q.astype(jnp.bfloat16), pV,
                         preferred_element_type=jnp.float32)
            kr = jnp.dot(k.astype(jnp.bfloat16), pV,
                         preferred_element_type=jnp.float32)
            q16 = (q * cosV + qr * sinV).astype(jnp.bfloat16)
            k16 = (k * cosV + kr * sinV).astype(jnp.bfloat16)
            for b in range(B):
                for t in range(Hl):
                    rs = slice(b * Sq, (b + 1) * Sq)
                    cs = slice(t * DH, (t + 1) * DH)
                    s = lax.dot_general(q16[rs, cs], k16[rs, cs],
                                        (((1,), (1,)), ((), ())),
                                        preferred_element_type=jnp.float32)
                    s = s * 0.125
                    m = jnp.max(s, axis=-1, keepdims=True)
                    e = jnp.exp(s - m)
                    wgt = (e / jnp.sum(e, axis=-1, keepdims=True)
                           ).astype(jnp.bfloat16)
                    ctx_ref[rs, cs] = jnp.dot(
                        wgt, v16[rs, cs],
                        preferred_element_type=jnp.float32).astype(jnp.bfloat16)
            out_ref[...] += lax.dot_general(
                ctx_ref[...], wo_t, (((1,), (1,)), ((), ())),
                preferred_element_type=jnp.float32)

        def step(s, carry):
            cur = lax.rem(s, NSLOT)
            nxt = lax.rem(s + 1, NSLOT)
            rd_r = pltpu.make_async_remote_copy(
                src_ref=comm_r.at[cur], dst_ref=comm_r.at[nxt],
                send_sem=ss_r.at[cur], recv_sem=rs_r.at[nxt],
                device_id=(right,), device_id_type=pl.DeviceIdType.MESH)
            rd_l = pltpu.make_async_remote_copy(
                src_ref=comm_l.at[cur], dst_ref=comm_l.at[nxt],
                send_sem=ss_l.at[cur], recv_sem=rs_l.at[nxt],
                device_id=(left,), device_id_type=pl.DeviceIdType.MESH)

            @pl.when(s < N_R)
            def _():
                @pl.when(s >= NSLOT - 1)
                def _():
                    pl.semaphore_wait(credit_r, 1)
                rd_r.start()

            @pl.when(s < N_L)
            def _():
                @pl.when(s >= NSLOT - 1)
                def _():
                    pl.semaphore_wait(credit_l, 1)
                rd_l.start()

            compute_chunk(comm_r[cur])
            @pl.when((s >= 1) & (s <= N_L))
            def _():
                compute_chunk(comm_l[cur])

            @pl.when(s < N_R)
            def _():
                rd_r.wait()
                pl.semaphore_signal(credit_r, inc=1, device_id=(left,),
                                    device_id_type=pl.DeviceIdType.MESH)

            @pl.when(s < N_L)
            def _():
                rd_l.wait()
                pl.semaphore_signal(credit_l, inc=1, device_id=(right,),
                                    device_id_type=pl.DeviceIdType.MESH)
            return carry

        lax.fori_loop(0, N_R + 1, step, 0)
        pl.semaphore_wait(credit_r, NSLOT - 1)
        pl.semaphore_wait(credit_l, NSLOT - 1)

    succ_s = jnp.asarray(_SUCC)
    pred_s = jnp.asarray(_PRED)

    out = pl.pallas_call(
        body,
        out_shape=jax.ShapeDtypeStruct((R, D), jnp.float32),
        in_specs=[pl.BlockSpec(memory_space=pltpu.SMEM)] * 2
        + [pl.BlockSpec(memory_space=pltpu.VMEM)] * 5,
        out_specs=pl.BlockSpec(memory_space=pltpu.VMEM),
        scratch_shapes=[
            pltpu.VMEM((NSLOT, D, 4 * C), jnp.bfloat16),
            pltpu.VMEM((NSLOT, D, 4 * C), jnp.bfloat16),
            pltpu.VMEM((R, C), jnp.bfloat16),
            pltpu.SemaphoreType.DMA((NSLOT,)),
            pltpu.SemaphoreType.DMA((NSLOT,)),
            pltpu.SemaphoreType.DMA((NSLOT,)),
            pltpu.SemaphoreType.DMA((NSLOT,)),
            pltpu.SemaphoreType.REGULAR,
            pltpu.SemaphoreType.REGULAR,
        ],
        compiler_params=pltpu.CompilerParams(collective_id=0),
    )(succ_s, pred_s, x2, w_all, cosT, sinT, P)
    return out.reshape(B, Sq, D)
